# Initial kernel scaffold; baseline (speedup 1.0000x reference)
import jax
import jax.numpy as jnp
from jax.experimental import pallas as pl


def kernel(input_, W):
    raise NotImplementedError("write your pallas kernel here")


# speedup vs baseline: 3.5674x; 3.5674x over previous
import jax
import jax.numpy as jnp
from jax import lax
from jax.experimental import pallas as pl
from jax.experimental.pallas import tpu as pltpu
from jax.experimental.pallas import tpu_sc as plsc

N_V = 1000
N_D = 64
BATCH = 4096
HIST = 200

NC = 2
NS = 16
NW = NC * NS

B_TOTAL = BATCH * HIST
CHUNK = 128
N_CHUNKS = B_TOTAL // CHUNK
CHUNKS_PER_W = N_CHUNKS // NW
NBUF = 2


def _embed_body(idx_hbm, table_hbm, out_hbm, idx_v, rows_v, sems):
  wid = lax.axis_index("s") * NC + lax.axis_index("c")
  chunk_base = wid * CHUNKS_PER_W

  pltpu.sync_copy(idx_hbm.at[pl.ds(chunk_base, CHUNKS_PER_W)], idx_v)

  def start_gather(j, buf):
    return pltpu.async_copy(
        table_hbm.at[idx_v.at[j]], rows_v.at[buf], sems.at[buf])

  def write_out(j, buf):
    row0 = (chunk_base + j) * CHUNK
    pltpu.sync_copy(rows_v.at[buf], out_hbm.at[pl.ds(row0, CHUNK)])

  start_gather(0, 0)

  def body(j, _):
    buf = lax.rem(j, NBUF)
    nxt = lax.rem(j + 1, NBUF)

    @pl.when(j + 1 < CHUNKS_PER_W)
    def _():
      start_gather(j + 1, nxt)

    pltpu.make_async_copy(
        table_hbm.at[idx_v.at[j]], rows_v.at[buf], sems.at[buf]).wait()
    write_out(j, buf)
    return 0

  lax.fori_loop(0, CHUNKS_PER_W, body, 0)


@jax.jit
def kernel(input_, W):
  idx2d = input_.reshape(N_CHUNKS, CHUNK)
  run = pl.kernel(
      _embed_body,
      out_type=jax.ShapeDtypeStruct((B_TOTAL, N_D), jnp.float32),
      mesh=plsc.VectorSubcoreMesh(core_axis_name="c", subcore_axis_name="s"),
      scratch_types=[
          pltpu.VMEM((CHUNKS_PER_W, CHUNK), jnp.int32),
          pltpu.VMEM((NBUF, CHUNK, N_D), jnp.float32),
          pltpu.SemaphoreType.DMA((NBUF,)),
      ],
      compiler_params=pltpu.CompilerParams(use_tc_tiling_on_sc=False),
  )
  out = run(idx2d, W)
  return out.reshape(BATCH, HIST, N_D)


# speedup vs baseline: 3.5831x; 1.0044x over previous
import jax
import jax.numpy as jnp
from jax import lax
from jax.experimental import pallas as pl
from jax.experimental.pallas import tpu as pltpu
from jax.experimental.pallas import tpu_sc as plsc

N_V = 1000
N_D = 64
BATCH = 4096
HIST = 200

NC = 2
NS = 16
NW = NC * NS

B_TOTAL = BATCH * HIST
CHUNK = 128
K = 4
GROUP = CHUNK * K
N_CHUNKS = B_TOTAL // CHUNK
CHUNKS_PER_W = N_CHUNKS // NW
N_GROUPS = CHUNKS_PER_W // K


def _embed_body(idx_hbm, table_hbm, out_hbm, idx_v, rows_v, gsems, wsems):
  wid = lax.axis_index("s") * NC + lax.axis_index("c")
  chunk_base = wid * CHUNKS_PER_W
  row_base = chunk_base * CHUNK

  pltpu.sync_copy(idx_hbm.at[pl.ds(chunk_base, CHUNKS_PER_W)], idx_v)

  def fire_group(g, pg):
    for b in range(K):
      pltpu.async_copy(
          table_hbm.at[idx_v.at[g * K + b]],
          rows_v.at[pg].at[pl.ds(b * CHUNK, CHUNK)],
          gsems.at[pg])

  def drain_group(g, pg):
    for b in range(K):
      pltpu.make_async_copy(
          table_hbm.at[idx_v.at[g * K + b]],
          rows_v.at[pg].at[pl.ds(b * CHUNK, CHUNK)],
          gsems.at[pg]).wait()

  def start_write(g, pg):
    pltpu.async_copy(
        rows_v.at[pg], out_hbm.at[pl.ds(row_base + g * GROUP, GROUP)],
        wsems.at[pg])

  def wait_write(g, pg):
    pltpu.make_async_copy(
        rows_v.at[pg], out_hbm.at[pl.ds(row_base + g * GROUP, GROUP)],
        wsems.at[pg]).wait()

  fire_group(0, 0)

  def body(g, _):
    pg = lax.rem(g, 2)
    og = 1 - pg

    @pl.when(g + 1 < N_GROUPS)
    def _():
      @pl.when(g >= 1)
      def _():
        wait_write(g - 1, og)
      fire_group(g + 1, og)

    drain_group(g, pg)
    start_write(g, pg)
    return 0

  lax.fori_loop(0, N_GROUPS, body, 0)

  wait_write(N_GROUPS - 2, lax.rem(N_GROUPS - 2, 2))
  wait_write(N_GROUPS - 1, lax.rem(N_GROUPS - 1, 2))


@jax.jit
def kernel(input_, W):
  idx2d = input_.reshape(N_CHUNKS, CHUNK)
  run = pl.kernel(
      _embed_body,
      out_type=jax.ShapeDtypeStruct((B_TOTAL, N_D), jnp.float32),
      mesh=plsc.VectorSubcoreMesh(core_axis_name="c", subcore_axis_name="s"),
      scratch_types=[
          pltpu.VMEM((CHUNKS_PER_W, CHUNK), jnp.int32),
          pltpu.VMEM((2, GROUP, N_D), jnp.float32),
          pltpu.SemaphoreType.DMA((2,)),
          pltpu.SemaphoreType.DMA((2,)),
      ],
      compiler_params=pltpu.CompilerParams(use_tc_tiling_on_sc=False),
  )
  out = run(idx2d, W)
  return out.reshape(BATCH, HIST, N_D)


# speedup vs baseline: 4.9939x; 1.3938x over previous
import jax
import jax.numpy as jnp
from jax import lax
from jax.experimental import pallas as pl
from jax.experimental.pallas import tpu as pltpu
from jax.experimental.pallas import tpu_sc as plsc

N_V = 1000
N_D = 64
BATCH = 4096
HIST = 200

NC = 2
NS = 16
NW = NC * NS

B_TOTAL = BATCH * HIST
CHUNK = 128
K = 4
GROUP = CHUNK * K
N_CHUNKS = B_TOTAL // CHUNK
CHUNKS_PER_W = N_CHUNKS // NW
N_GROUPS = CHUNKS_PER_W // K


def _embed_body(idx_hbm, table_hbm, out_hbm, idx_v, rows_v, table_sh,
                gsems, wsems):
  wid = lax.axis_index("s") * NC + lax.axis_index("c")
  chunk_base = wid * CHUNKS_PER_W
  row_base = chunk_base * CHUNK

  @pl.when(lax.axis_index("s") == 0)
  def _():
    pltpu.sync_copy(table_hbm, table_sh)

  pltpu.sync_copy(idx_hbm.at[pl.ds(chunk_base, CHUNKS_PER_W)], idx_v)
  plsc.subcore_barrier()

  def fire_group(g, pg):
    for b in range(K):
      pltpu.async_copy(
          table_sh.at[idx_v.at[g * K + b]],
          rows_v.at[pg].at[pl.ds(b * CHUNK, CHUNK)],
          gsems.at[pg])

  def drain_group(g, pg):
    for b in range(K):
      pltpu.make_async_copy(
          table_sh.at[idx_v.at[g * K + b]],
          rows_v.at[pg].at[pl.ds(b * CHUNK, CHUNK)],
          gsems.at[pg]).wait()

  def start_write(g, pg):
    pltpu.async_copy(
        rows_v.at[pg], out_hbm.at[pl.ds(row_base + g * GROUP, GROUP)],
        wsems.at[pg])

  def wait_write(g, pg):
    pltpu.make_async_copy(
        rows_v.at[pg], out_hbm.at[pl.ds(row_base + g * GROUP, GROUP)],
        wsems.at[pg]).wait()

  fire_group(0, 0)

  def body(g, _):
    pg = lax.rem(g, 2)
    og = 1 - pg

    @pl.when(g + 1 < N_GROUPS)
    def _():
      @pl.when(g >= 1)
      def _():
        wait_write(g - 1, og)
      fire_group(g + 1, og)

    drain_group(g, pg)
    start_write(g, pg)
    return 0

  lax.fori_loop(0, N_GROUPS, body, 0)

  wait_write(N_GROUPS - 2, lax.rem(N_GROUPS - 2, 2))
  wait_write(N_GROUPS - 1, lax.rem(N_GROUPS - 1, 2))


@jax.jit
def kernel(input_, W):
  idx2d = input_.reshape(N_CHUNKS, CHUNK)
  run = pl.kernel(
      _embed_body,
      out_type=jax.ShapeDtypeStruct((B_TOTAL, N_D), jnp.float32),
      mesh=plsc.VectorSubcoreMesh(core_axis_name="c", subcore_axis_name="s"),
      scratch_types=[
          pltpu.VMEM((CHUNKS_PER_W, CHUNK), jnp.int32),
          pltpu.VMEM((2, GROUP, N_D), jnp.float32),
          pltpu.VMEM_SHARED((N_V, N_D), jnp.float32),
          pltpu.SemaphoreType.DMA((2,)),
          pltpu.SemaphoreType.DMA((2,)),
      ],
      compiler_params=pltpu.CompilerParams(use_tc_tiling_on_sc=False),
  )
  out = run(idx2d, W)
  return out.reshape(BATCH, HIST, N_D)
